# Initial kernel scaffold; baseline (speedup 1.0000x reference)
import jax
import jax.numpy as jnp
from jax.experimental import pallas as pl


def kernel(x):
    raise NotImplementedError("write your pallas kernel here")


# speedup vs baseline: 13.9899x; 13.9899x over previous
import jax
import jax.numpy as jnp
from jax import lax
from jax.experimental import pallas as pl
from jax.experimental.pallas import tpu as pltpu
from jax.experimental.pallas import tpu_sc as plsc

_L = 16
_NW = 32

_SORT8 = [(0, 1), (2, 3), (4, 5), (6, 7),
          (0, 2), (1, 3), (4, 6), (5, 7),
          (1, 2), (5, 6), (0, 4), (3, 7),
          (1, 5), (2, 6),
          (1, 4), (3, 6),
          (2, 4), (3, 5),
          (3, 4)]

_BITONIC8 = [(0, 4), (1, 5), (2, 6), (3, 7),
             (0, 2), (1, 3), (4, 6), (5, 7),
             (0, 1), (2, 3), (4, 5), (6, 7)]


def _ce(v, a, b):
    hi = jnp.maximum(v[a], v[b])
    lo = jnp.minimum(v[a], v[b])
    v[a] = hi
    v[b] = lo


def _make_sc_call(B, CD, HW):
    assert CD % 8 == 0 and HW % _L == 0
    gpb = HW // _L
    ngroups = B * gpb
    nblk = CD // 8

    def body(x_hbm, tk_hbm, mn_hbm, buf, tkv, mnv):
        nc = plsc.get_sparse_core_info().num_cores
        wid = lax.axis_index("s") * nc + lax.axis_index("c")
        ng = (ngroups - wid + _NW - 1) // _NW

        def group_body(t, carry):
            g = wid + t * _NW
            b = g // gpb
            p0 = (g % gpb) * _L
            pltpu.sync_copy(x_hbm.at[b, :, pl.ds(p0, _L)], buf)

            def blk_body(i, c):
                acc = list(c[:8])
                tot = c[8]
                v = [buf[8 * i + j, :] for j in range(8)]
                tot = tot + (((v[0] + v[1]) + (v[2] + v[3]))
                             + ((v[4] + v[5]) + (v[6] + v[7])))
                for a, bb in _SORT8:
                    _ce(v, a, bb)
                m = [jnp.maximum(acc[j], v[7 - j]) for j in range(8)]
                for a, bb in _BITONIC8:
                    _ce(m, a, bb)
                return (*m, tot)

            ninf = jnp.full((_L,), -jnp.inf, jnp.float32)
            zero = jnp.zeros((_L,), jnp.float32)
            out = lax.fori_loop(0, nblk, blk_body, (ninf,) * 8 + (zero,))
            tk = ((out[0] + out[1]) + (out[2] + out[3])) + \
                 ((out[4] + out[5]) + (out[6] + out[7]))
            tkv[...] = tk
            mnv[...] = out[8] * (1.0 / CD)
            pltpu.sync_copy(tkv, tk_hbm.at[b, pl.ds(p0, _L)])
            pltpu.sync_copy(mnv, mn_hbm.at[b, pl.ds(p0, _L)])
            return carry

        lax.fori_loop(0, ng, group_body, 0)

    mesh = plsc.VectorSubcoreMesh(core_axis_name="c", subcore_axis_name="s")
    return pl.kernel(
        body,
        out_type=[jax.ShapeDtypeStruct((B, HW), jnp.float32),
                  jax.ShapeDtypeStruct((B, HW), jnp.float32)],
        mesh=mesh,
        compiler_params=pltpu.CompilerParams(use_tc_tiling_on_sc=False),
        scratch_types=[pltpu.VMEM((CD, _L), jnp.float32),
                       pltpu.VMEM((_L,), jnp.float32),
                       pltpu.VMEM((_L,), jnp.float32)],
    )


def kernel(x):
    b, c, d, h, w = x.shape
    x3 = x.reshape(b, c * d, h * w)
    tk, mn = _make_sc_call(b, c * d, h * w)(x3)
    return (tk.reshape(b, 1, 1, h, w), mn.reshape(b, 1, 1, h, w))
